# baseline (speedup 1.0000x reference)
import jax
import jax.numpy as jnp
from jax import lax
from jax.experimental import pallas as pl
from jax.experimental.pallas import tpu as pltpu
from jax.experimental.pallas import tpu_sc as plsc

N = 10000
NPAD = 10240
E = 160000
NG = 64
NGP = 72
NC = 2
NS = 16
CH = 128
NCHUNK = 80
EPT = CH * NCHUNK
EPAD = EPT * NS
RPT = NPAD // NS
ZOFF = NPAD - CH
DEGC = NCHUNK // NC


def _deg_body(dst_hbm, aux_hbm, out_hbm, dst_v, ones_v, deg_sh, dsem):
    c = lax.axis_index("c")
    s = lax.axis_index("s")
    pltpu.sync_copy(dst_hbm.at[s, pl.ds(c * DEGC, DEGC)], dst_v)
    pltpu.sync_copy(aux_hbm.at[pl.ds(0, CH)], ones_v)
    pltpu.sync_copy(aux_hbm.at[pl.ds(CH, RPT)], deg_sh.at[pl.ds(s * RPT, RPT)])
    plsc.subcore_barrier()

    @pl.loop(0, DEGC, step=8)
    def _chunk(j0):
        for k in range(8):
            pltpu.async_copy(ones_v, deg_sh.at[dst_v.at[j0 + k]], dsem,
                             add=True)
        for k in range(8):
            pltpu.make_async_copy(ones_v, deg_sh.at[dst_v.at[j0]], dsem).wait()

    plsc.subcore_barrier()
    pltpu.sync_copy(deg_sh.at[pl.ds(s * RPT, RPT)],
                    out_hbm.at[c, pl.ds(s * RPT, RPT)])


_deg_call = pl.kernel(
    _deg_body,
    out_type=jax.ShapeDtypeStruct((NC, NPAD), jnp.float32),
    mesh=plsc.VectorSubcoreMesh(core_axis_name="c", subcore_axis_name="s"),
    scratch_types=[
        pltpu.VMEM((DEGC, CH), jnp.int32),
        pltpu.VMEM((CH,), jnp.float32),
        pltpu.VMEM_SHARED((NPAD,), jnp.float32),
        pltpu.SemaphoreType.DMA,
    ],
)


def _agg_pipeline(xs_hbm, src_hbm, dst_hbm, src_v, dst_v, rows_v, agg_sh,
                  xs_sh, gsem, ssem, c, s, F2, nbuf):
    pltpu.sync_copy(src_hbm.at[s], src_v)
    pltpu.sync_copy(dst_hbm.at[s], dst_v)
    pltpu.sync_copy(xs_hbm.at[pl.ds(s * RPT, RPT), pl.ds(c * F2, F2)],
                    xs_sh.at[pl.ds(s * RPT, RPT)])
    for k in range(RPT // CH):
        pltpu.sync_copy(xs_hbm.at[pl.ds(ZOFF, CH), pl.ds(c * F2, F2)],
                        agg_sh.at[pl.ds(s * RPT + k * CH, CH)])
    plsc.subcore_barrier()

    def _start_gather(j, b):
        pltpu.async_copy(xs_sh.at[src_v.at[j]], rows_v.at[b], gsem.at[b])

    def _wait_gather(b):
        pltpu.make_async_copy(xs_sh.at[src_v.at[0]], rows_v.at[b],
                              gsem.at[b]).wait()

    def _start_scatter(j, b):
        pltpu.async_copy(rows_v.at[b], agg_sh.at[dst_v.at[j]], ssem.at[b],
                         add=True)

    def _wait_scatter(b):
        pltpu.make_async_copy(rows_v.at[b], agg_sh.at[dst_v.at[0]],
                              ssem.at[b]).wait()

    for b in range(nbuf):
        _start_gather(b, b)

    @pl.loop(0, NCHUNK - nbuf, step=nbuf)
    def _round(j0):
        for b in range(nbuf):
            _wait_gather(b)
            _start_scatter(j0 + b, b)
        for b in range(nbuf):
            _wait_scatter(b)
            _start_gather(j0 + b + nbuf, b)

    for b in range(nbuf):
        _wait_gather(b)
        _start_scatter(NCHUNK - nbuf + b, b)
    for b in range(nbuf):
        _wait_scatter(b)

    plsc.subcore_barrier()


def _agg32_body(xs_hbm, src_hbm, dst_hbm, out_hbm, src_v, dst_v, rows_v,
                agg_sh, xs_sh, gsem, ssem):
    c = lax.axis_index("c")
    s = lax.axis_index("s")
    _agg_pipeline(xs_hbm, src_hbm, dst_hbm, src_v, dst_v, rows_v, agg_sh,
                  xs_sh, gsem, ssem, c, s, 16, 8)
    pltpu.sync_copy(agg_sh.at[pl.ds(s * RPT, RPT)],
                    out_hbm.at[pl.ds(s * RPT, RPT), pl.ds(c * 16, 16)])


_agg32_call = pl.kernel(
    _agg32_body,
    out_type=jax.ShapeDtypeStruct((NPAD, 32), jnp.float32),
    mesh=plsc.VectorSubcoreMesh(core_axis_name="c", subcore_axis_name="s"),
    compiler_params=pltpu.CompilerParams(use_tc_tiling_on_sc=False),
    scratch_types=[
        pltpu.VMEM((NCHUNK, CH), jnp.int32),
        pltpu.VMEM((NCHUNK, CH), jnp.int32),
        pltpu.VMEM((8, CH, 16), jnp.float32),
        pltpu.VMEM_SHARED((NPAD, 16), jnp.float32),
        pltpu.VMEM_SHARED((NPAD, 16), jnp.float32),
        pltpu.SemaphoreType.DMA((8,)),
        pltpu.SemaphoreType.DMA((8,)),
    ],
)


def _agg64_body(xs_hbm, src_hbm, dst_hbm, xw2_hbm, dinv_hbm, b2_hbm,
                batch_hbm, out_hbm, src_v, dst_v, rows_v, agg_sh, xs_sh,
                gsem, ssem, agg_v, xw_v, dinv_v, batch_v, b2_v, acc_v):
    F2 = 32
    c = lax.axis_index("c")
    s = lax.axis_index("s")
    _agg_pipeline(xs_hbm, src_hbm, dst_hbm, src_v, dst_v, rows_v, agg_sh,
                  xs_sh, gsem, ssem, c, s, F2, 4)

    pltpu.sync_copy(agg_sh.at[pl.ds(s * RPT, RPT)], agg_v)
    pltpu.sync_copy(xw2_hbm.at[pl.ds(s * RPT, RPT), pl.ds(c * F2, F2)], xw_v)
    pltpu.sync_copy(dinv_hbm.at[pl.ds(s * RPT, RPT)], dinv_v)
    pltpu.sync_copy(batch_hbm.at[pl.ds(s * RPT, RPT)], batch_v)
    pltpu.sync_copy(b2_hbm.at[c], b2_v)

    neg = jnp.full((16,), -jnp.inf, dtype=jnp.float32)

    @pl.loop(0, NGP)
    def _init(g):
        for k in range(F2 // 16):
            acc_v[g, pl.ds(k * 16, 16)] = neg

    b2a = b2_v[pl.ds(0, 16)]
    b2b = b2_v[pl.ds(16, 16)]

    @pl.loop(0, RPT, step=16)
    def _row(r0):
        d16 = dinv_v[pl.ds(r0, 16)]
        g16 = batch_v[pl.ds(r0, 16)]
        for i in range(16):
            r = r0 + i
            d = d16[i]
            dd = d * d
            g = g16[i]
            ha = d * agg_v[r, pl.ds(0, 16)] + dd * xw_v[r, pl.ds(0, 16)] + b2a
            hb = d * agg_v[r, pl.ds(16, 16)] + dd * xw_v[r, pl.ds(16, 16)] + b2b
            acc_v[g, pl.ds(0, 16)] = jnp.maximum(acc_v[g, pl.ds(0, 16)], ha)
            acc_v[g, pl.ds(16, 16)] = jnp.maximum(acc_v[g, pl.ds(16, 16)], hb)

    pltpu.sync_copy(acc_v, out_hbm.at[c, s])


_agg64_call = pl.kernel(
    _agg64_body,
    out_type=jax.ShapeDtypeStruct((NC, NS, NGP, 32), jnp.float32),
    mesh=plsc.VectorSubcoreMesh(core_axis_name="c", subcore_axis_name="s"),
    compiler_params=pltpu.CompilerParams(use_tc_tiling_on_sc=False),
    scratch_types=[
        pltpu.VMEM((NCHUNK, CH), jnp.int32),
        pltpu.VMEM((NCHUNK, CH), jnp.int32),
        pltpu.VMEM((4, CH, 32), jnp.float32),
        pltpu.VMEM_SHARED((NPAD, 32), jnp.float32),
        pltpu.VMEM_SHARED((NPAD, 32), jnp.float32),
        pltpu.SemaphoreType.DMA((4,)),
        pltpu.SemaphoreType.DMA((4,)),
        pltpu.VMEM((RPT, 32), jnp.float32),
        pltpu.VMEM((RPT, 32), jnp.float32),
        pltpu.VMEM((RPT,), jnp.float32),
        pltpu.VMEM((RPT,), jnp.int32),
        pltpu.VMEM((32,), jnp.float32),
        pltpu.VMEM((NGP, 32), jnp.float32),
    ],
)


def _tc1_body(x_ref, w1_ref, p_ref, xw1_ref, dinv_ref, xs1_ref):
    xw1 = jnp.dot(x_ref[...], w1_ref[...], preferred_element_type=jnp.float32)
    p = p_ref[...]
    deg = p[0] + p[1] + 1.0
    dinv = lax.rsqrt(deg)[:, None]
    row = lax.broadcasted_iota(jnp.int32, (NPAD, 1), 0)
    dinv = jnp.where(row < N, dinv, 0.0)
    xw1_ref[...] = xw1
    dinv_ref[...] = dinv
    xs1_ref[...] = xw1 * dinv


_tc1 = pl.pallas_call(
    _tc1_body,
    out_shape=(
        jax.ShapeDtypeStruct((NPAD, 32), jnp.float32),
        jax.ShapeDtypeStruct((NPAD, 1), jnp.float32),
        jax.ShapeDtypeStruct((NPAD, 32), jnp.float32),
    ),
)


def _tc2_body(q_ref, xw1_ref, dinv_ref, b1_ref, w2_ref, xw2_ref, xs2_ref):
    agg1 = q_ref[...]
    dinv = dinv_ref[...]
    pre = dinv * agg1 + dinv * dinv * xw1_ref[...] + b1_ref[...]
    h1 = jnp.where(pre >= 0, pre, 0.1 * pre)
    xw2 = jnp.dot(h1, w2_ref[...], preferred_element_type=jnp.float32)
    xw2_ref[...] = xw2
    xs2_ref[...] = xw2 * dinv


_tc2 = pl.pallas_call(
    _tc2_body,
    out_shape=(
        jax.ShapeDtypeStruct((NPAD, 64), jnp.float32),
        jax.ShapeDtypeStruct((NPAD, 64), jnp.float32),
    ),
)


def _tc3_body(m_ref, l1w_ref, l1b_ref, l2w_ref, l2b_ref, y_ref):
    m = m_ref[...]
    pooled = jnp.concatenate([jnp.max(m[0, :, :NG, :], axis=0),
                              jnp.max(m[1, :, :NG, :], axis=0)], axis=-1)
    t = jnp.dot(pooled, l1w_ref[...], preferred_element_type=jnp.float32) + l1b_ref[...]
    t = jnp.where(t >= 0, t, 0.1 * t)
    y_ref[...] = jnp.dot(t, l2w_ref[...], preferred_element_type=jnp.float32) + l2b_ref[...]


_tc3 = pl.pallas_call(
    _tc3_body,
    out_shape=jax.ShapeDtypeStruct((NG, 32), jnp.float32),
)


def kernel(x, edge_index, batch, W1, b1, W2, b2, L1W, L1b, L2W, L2b):
    edge_index = edge_index.astype(jnp.int32)
    src = edge_index[0]
    dst = edge_index[1]
    pad = jnp.full((EPAD - E,), N, dtype=jnp.int32)
    src_r = jnp.concatenate([src, pad]).reshape(NS, NCHUNK, CH)
    dst_r = jnp.concatenate([dst, pad]).reshape(NS, NCHUNK, CH)
    aux = jnp.concatenate([jnp.ones((CH,), jnp.float32),
                           jnp.zeros((RPT,), jnp.float32)])
    x_pad = jnp.pad(x, ((0, NPAD - N), (0, 0)))
    batch_pad = jnp.concatenate(
        [batch.astype(jnp.int32), jnp.full((NPAD - N,), NG, jnp.int32)])
    b2_r = b2.reshape(NC, 32)

    p = _deg_call(dst_r, aux)
    xw1, dinv, xs1 = _tc1(x_pad, W1, p)
    q = _agg32_call(xs1, src_r, dst_r)
    xw2, xs2 = _tc2(q, xw1, dinv, b1, W2)
    m = _agg64_call(xs2, src_r, dst_r, xw2, dinv[:, 0], b2_r, batch_pad)
    y = _tc3(m, L1W, L1b, L2W, L2b)
    return y


# speedup vs baseline: 1.0522x; 1.0522x over previous
import jax
import jax.numpy as jnp
from jax import lax
from jax.experimental import pallas as pl
from jax.experimental.pallas import tpu as pltpu
from jax.experimental.pallas import tpu_sc as plsc

N = 10000
NPAD = 10240
E = 160000
NG = 64
NGP = 72
NC = 2
NS = 16
CH = 128
NCHUNK = 80
EPT = CH * NCHUNK
EPAD = EPT * NS
RPT = NPAD // NS
ZOFF = NPAD - CH
DEGC = NCHUNK // NC


def _deg_body(dst_hbm, aux_hbm, out_hbm, dst_v, ones_v, deg_sh, dsem):
    c = lax.axis_index("c")
    s = lax.axis_index("s")
    pltpu.sync_copy(dst_hbm.at[s, pl.ds(c * DEGC, DEGC)], dst_v)
    pltpu.sync_copy(aux_hbm.at[pl.ds(0, CH)], ones_v)
    pltpu.sync_copy(aux_hbm.at[pl.ds(CH, RPT)], deg_sh.at[pl.ds(s * RPT, RPT)])
    plsc.subcore_barrier()

    @pl.loop(0, DEGC, step=8)
    def _chunk(j0):
        for k in range(8):
            pltpu.async_copy(ones_v, deg_sh.at[dst_v.at[j0 + k]], dsem,
                             add=True)
        for k in range(8):
            pltpu.make_async_copy(ones_v, deg_sh.at[dst_v.at[j0]], dsem).wait()

    plsc.subcore_barrier()
    pltpu.sync_copy(deg_sh.at[pl.ds(s * RPT, RPT)],
                    out_hbm.at[c, pl.ds(s * RPT, RPT)])


_deg_call = pl.kernel(
    _deg_body,
    out_type=jax.ShapeDtypeStruct((NC, NPAD), jnp.float32),
    mesh=plsc.VectorSubcoreMesh(core_axis_name="c", subcore_axis_name="s"),
    scratch_types=[
        pltpu.VMEM((DEGC, CH), jnp.int32),
        pltpu.VMEM((CH,), jnp.float32),
        pltpu.VMEM_SHARED((NPAD,), jnp.float32),
        pltpu.SemaphoreType.DMA,
    ],
)


def _agg_pipeline(xs_hbm, src_hbm, dst_hbm, src_v, dst_v, rows_v, agg_sh,
                  xs_sh, gsem, ssem, c, s, F2, nbuf):
    pltpu.sync_copy(src_hbm.at[s], src_v)
    pltpu.sync_copy(dst_hbm.at[s], dst_v)
    pltpu.sync_copy(xs_hbm.at[pl.ds(s * RPT, RPT), pl.ds(c * F2, F2)],
                    xs_sh.at[pl.ds(s * RPT, RPT)])
    for k in range(RPT // CH):
        pltpu.sync_copy(xs_hbm.at[pl.ds(ZOFF, CH), pl.ds(c * F2, F2)],
                        agg_sh.at[pl.ds(s * RPT + k * CH, CH)])
    plsc.subcore_barrier()

    def _start_gather(j, b):
        pltpu.async_copy(xs_sh.at[src_v.at[j]], rows_v.at[b], gsem.at[b])

    def _wait_gather(b):
        pltpu.make_async_copy(xs_sh.at[src_v.at[0]], rows_v.at[b],
                              gsem.at[b]).wait()

    def _start_scatter(j, b):
        pltpu.async_copy(rows_v.at[b], agg_sh.at[dst_v.at[j]], ssem.at[b],
                         add=True)

    def _wait_scatter(b):
        pltpu.make_async_copy(rows_v.at[b], agg_sh.at[dst_v.at[0]],
                              ssem.at[b]).wait()

    for b in range(nbuf):
        _start_gather(b, b)

    @pl.loop(0, NCHUNK - nbuf, step=nbuf)
    def _round(j0):
        for b in range(nbuf):
            _wait_gather(b)
            _start_scatter(j0 + b, b)
            _wait_scatter(b)
            _start_gather(j0 + b + nbuf, b)

    for b in range(nbuf):
        _wait_gather(b)
        _start_scatter(NCHUNK - nbuf + b, b)
        _wait_scatter(b)

    plsc.subcore_barrier()


def _agg32_body(xs_hbm, src_hbm, dst_hbm, out_hbm, src_v, dst_v, rows_v,
                agg_sh, xs_sh, gsem, ssem):
    c = lax.axis_index("c")
    s = lax.axis_index("s")
    _agg_pipeline(xs_hbm, src_hbm, dst_hbm, src_v, dst_v, rows_v, agg_sh,
                  xs_sh, gsem, ssem, c, s, 16, 8)
    pltpu.sync_copy(agg_sh.at[pl.ds(s * RPT, RPT)],
                    out_hbm.at[pl.ds(s * RPT, RPT), pl.ds(c * 16, 16)])


_agg32_call = pl.kernel(
    _agg32_body,
    out_type=jax.ShapeDtypeStruct((NPAD, 32), jnp.float32),
    mesh=plsc.VectorSubcoreMesh(core_axis_name="c", subcore_axis_name="s"),
    compiler_params=pltpu.CompilerParams(use_tc_tiling_on_sc=False),
    scratch_types=[
        pltpu.VMEM((NCHUNK, CH), jnp.int32),
        pltpu.VMEM((NCHUNK, CH), jnp.int32),
        pltpu.VMEM((8, CH, 16), jnp.float32),
        pltpu.VMEM_SHARED((NPAD, 16), jnp.float32),
        pltpu.VMEM_SHARED((NPAD, 16), jnp.float32),
        pltpu.SemaphoreType.DMA((8,)),
        pltpu.SemaphoreType.DMA((8,)),
    ],
)


def _agg64_body(xs_hbm, src_hbm, dst_hbm, xw2_hbm, dinv_hbm, b2_hbm,
                batch_hbm, out_hbm, src_v, dst_v, rows_v, agg_sh, xs_sh,
                gsem, ssem, agg_v, xw_v, dinv_v, batch_v, b2_v, acc_v):
    F2 = 32
    c = lax.axis_index("c")
    s = lax.axis_index("s")
    _agg_pipeline(xs_hbm, src_hbm, dst_hbm, src_v, dst_v, rows_v, agg_sh,
                  xs_sh, gsem, ssem, c, s, F2, 4)

    pltpu.sync_copy(agg_sh.at[pl.ds(s * RPT, RPT)], agg_v)
    pltpu.sync_copy(xw2_hbm.at[pl.ds(s * RPT, RPT), pl.ds(c * F2, F2)], xw_v)
    pltpu.sync_copy(dinv_hbm.at[pl.ds(s * RPT, RPT)], dinv_v)
    pltpu.sync_copy(batch_hbm.at[pl.ds(s * RPT, RPT)], batch_v)
    pltpu.sync_copy(b2_hbm.at[c], b2_v)

    neg = jnp.full((16,), -jnp.inf, dtype=jnp.float32)

    @pl.loop(0, NGP)
    def _init(g):
        for k in range(F2 // 16):
            acc_v[g, pl.ds(k * 16, 16)] = neg

    b2a = b2_v[pl.ds(0, 16)]
    b2b = b2_v[pl.ds(16, 16)]

    @pl.loop(0, RPT, step=16)
    def _row(r0):
        d16 = dinv_v[pl.ds(r0, 16)]
        g16 = batch_v[pl.ds(r0, 16)]
        for i in range(16):
            r = r0 + i
            d = d16[i]
            dd = d * d
            g = g16[i]
            ha = d * agg_v[r, pl.ds(0, 16)] + dd * xw_v[r, pl.ds(0, 16)] + b2a
            hb = d * agg_v[r, pl.ds(16, 16)] + dd * xw_v[r, pl.ds(16, 16)] + b2b
            acc_v[g, pl.ds(0, 16)] = jnp.maximum(acc_v[g, pl.ds(0, 16)], ha)
            acc_v[g, pl.ds(16, 16)] = jnp.maximum(acc_v[g, pl.ds(16, 16)], hb)

    pltpu.sync_copy(acc_v, out_hbm.at[c, s])


_agg64_call = pl.kernel(
    _agg64_body,
    out_type=jax.ShapeDtypeStruct((NC, NS, NGP, 32), jnp.float32),
    mesh=plsc.VectorSubcoreMesh(core_axis_name="c", subcore_axis_name="s"),
    compiler_params=pltpu.CompilerParams(use_tc_tiling_on_sc=False),
    scratch_types=[
        pltpu.VMEM((NCHUNK, CH), jnp.int32),
        pltpu.VMEM((NCHUNK, CH), jnp.int32),
        pltpu.VMEM((4, CH, 32), jnp.float32),
        pltpu.VMEM_SHARED((NPAD, 32), jnp.float32),
        pltpu.VMEM_SHARED((NPAD, 32), jnp.float32),
        pltpu.SemaphoreType.DMA((4,)),
        pltpu.SemaphoreType.DMA((4,)),
        pltpu.VMEM((RPT, 32), jnp.float32),
        pltpu.VMEM((RPT, 32), jnp.float32),
        pltpu.VMEM((RPT,), jnp.float32),
        pltpu.VMEM((RPT,), jnp.int32),
        pltpu.VMEM((32,), jnp.float32),
        pltpu.VMEM((NGP, 32), jnp.float32),
    ],
)


def _tc1_body(x_ref, w1_ref, p_ref, xw1_ref, dinv_ref, xs1_ref):
    xw1 = jnp.dot(x_ref[...], w1_ref[...], preferred_element_type=jnp.float32)
    p = p_ref[...]
    deg = p[0] + p[1] + 1.0
    dinv = lax.rsqrt(deg)[:, None]
    row = lax.broadcasted_iota(jnp.int32, (NPAD, 1), 0)
    dinv = jnp.where(row < N, dinv, 0.0)
    xw1_ref[...] = xw1
    dinv_ref[...] = dinv
    xs1_ref[...] = xw1 * dinv


_tc1 = pl.pallas_call(
    _tc1_body,
    out_shape=(
        jax.ShapeDtypeStruct((NPAD, 32), jnp.float32),
        jax.ShapeDtypeStruct((NPAD, 1), jnp.float32),
        jax.ShapeDtypeStruct((NPAD, 32), jnp.float32),
    ),
)


def _tc2_body(q_ref, xw1_ref, dinv_ref, b1_ref, w2_ref, xw2_ref, xs2_ref):
    agg1 = q_ref[...]
    dinv = dinv_ref[...]
    pre = dinv * agg1 + dinv * dinv * xw1_ref[...] + b1_ref[...]
    h1 = jnp.where(pre >= 0, pre, 0.1 * pre)
    xw2 = jnp.dot(h1, w2_ref[...], preferred_element_type=jnp.float32)
    xw2_ref[...] = xw2
    xs2_ref[...] = xw2 * dinv


_tc2 = pl.pallas_call(
    _tc2_body,
    out_shape=(
        jax.ShapeDtypeStruct((NPAD, 64), jnp.float32),
        jax.ShapeDtypeStruct((NPAD, 64), jnp.float32),
    ),
)


def _tc3_body(m_ref, l1w_ref, l1b_ref, l2w_ref, l2b_ref, y_ref):
    m = m_ref[...]
    pooled = jnp.concatenate([jnp.max(m[0, :, :NG, :], axis=0),
                              jnp.max(m[1, :, :NG, :], axis=0)], axis=-1)
    t = jnp.dot(pooled, l1w_ref[...], preferred_element_type=jnp.float32) + l1b_ref[...]
    t = jnp.where(t >= 0, t, 0.1 * t)
    y_ref[...] = jnp.dot(t, l2w_ref[...], preferred_element_type=jnp.float32) + l2b_ref[...]


_tc3 = pl.pallas_call(
    _tc3_body,
    out_shape=jax.ShapeDtypeStruct((NG, 32), jnp.float32),
)


def kernel(x, edge_index, batch, W1, b1, W2, b2, L1W, L1b, L2W, L2b):
    edge_index = edge_index.astype(jnp.int32)
    src = edge_index[0]
    dst = edge_index[1]
    pad = jnp.full((EPAD - E,), N, dtype=jnp.int32)
    src_r = jnp.concatenate([src, pad]).reshape(NS, NCHUNK, CH)
    dst_r = jnp.concatenate([dst, pad]).reshape(NS, NCHUNK, CH)
    aux = jnp.concatenate([jnp.ones((CH,), jnp.float32),
                           jnp.zeros((RPT,), jnp.float32)])
    x_pad = jnp.pad(x, ((0, NPAD - N), (0, 0)))
    batch_pad = jnp.concatenate(
        [batch.astype(jnp.int32), jnp.full((NPAD - N,), NG, jnp.int32)])
    b2_r = b2.reshape(NC, 32)

    p = _deg_call(dst_r, aux)
    xw1, dinv, xs1 = _tc1(x_pad, W1, p)
    q = _agg32_call(xs1, src_r, dst_r)
    xw2, xs2 = _tc2(q, xw1, dinv, b1, W2)
    m = _agg64_call(xs2, src_r, dst_r, xw2, dinv[:, 0], b2_r, batch_pad)
    y = _tc3(m, L1W, L1b, L2W, L2b)
    return y


# speedup vs baseline: 1.0700x; 1.0169x over previous
import jax
import jax.numpy as jnp
from jax import lax
from jax.experimental import pallas as pl
from jax.experimental.pallas import tpu as pltpu
from jax.experimental.pallas import tpu_sc as plsc

N = 10000
NPAD = 10240
E = 160000
NG = 64
NGP = 72
NC = 2
NS = 16
CH = 128
NCHUNK = 80
EPT = CH * NCHUNK
EPAD = EPT * NS
RPT = NPAD // NS
ZOFF = NPAD - CH
DEGC = NCHUNK // NC


def _deg_body(dst_hbm, aux_hbm, out_hbm, dst_v, ones_v, deg_sh, dsem):
    c = lax.axis_index("c")
    s = lax.axis_index("s")
    pltpu.sync_copy(dst_hbm.at[s, pl.ds(c * DEGC, DEGC)], dst_v)
    pltpu.sync_copy(aux_hbm.at[pl.ds(0, CH)], ones_v)
    pltpu.sync_copy(aux_hbm.at[pl.ds(CH, RPT)], deg_sh.at[pl.ds(s * RPT, RPT)])
    plsc.subcore_barrier()

    @pl.loop(0, DEGC, step=8)
    def _chunk(j0):
        for k in range(8):
            pltpu.async_copy(ones_v, deg_sh.at[dst_v.at[j0 + k]], dsem,
                             add=True)
        for k in range(8):
            pltpu.make_async_copy(ones_v, deg_sh.at[dst_v.at[j0]], dsem).wait()

    plsc.subcore_barrier()
    pltpu.sync_copy(deg_sh.at[pl.ds(s * RPT, RPT)],
                    out_hbm.at[c, pl.ds(s * RPT, RPT)])


_deg_call = pl.kernel(
    _deg_body,
    out_type=jax.ShapeDtypeStruct((NC, NPAD), jnp.float32),
    mesh=plsc.VectorSubcoreMesh(core_axis_name="c", subcore_axis_name="s"),
    scratch_types=[
        pltpu.VMEM((DEGC, CH), jnp.int32),
        pltpu.VMEM((CH,), jnp.float32),
        pltpu.VMEM_SHARED((NPAD,), jnp.float32),
        pltpu.SemaphoreType.DMA,
    ],
)


def _stage_scaled(xw_hbm, dinv_hbm, xw_v, dinv_v, xs_sh, c, s, F2):
    pltpu.sync_copy(xw_hbm.at[pl.ds(s * RPT, RPT), pl.ds(c * F2, F2)], xw_v)
    pltpu.sync_copy(dinv_hbm.at[pl.ds(s * RPT, RPT)], dinv_v)

    @pl.loop(0, RPT, step=16)
    def _scale(r0):
        d16 = dinv_v[pl.ds(r0, 16)]
        for i in range(16):
            d = d16[i]
            for k in range(F2 // 16):
                xw_v[r0 + i, pl.ds(k * 16, 16)] = (
                    d * xw_v[r0 + i, pl.ds(k * 16, 16)])

    pltpu.sync_copy(xw_v, xs_sh.at[pl.ds(s * RPT, RPT)])


def _agg_pipeline(xw_hbm, src_hbm, dst_hbm, src_v, dst_v, rows_v, agg_sh,
                  xs_sh, gsem, ssem, c, s, F2, nbuf):
    for k in range(RPT // CH):
        pltpu.sync_copy(xw_hbm.at[pl.ds(ZOFF, CH), pl.ds(c * F2, F2)],
                        agg_sh.at[pl.ds(s * RPT + k * CH, CH)])
    plsc.subcore_barrier()

    def _start_gather(j, b):
        pltpu.async_copy(xs_sh.at[src_v.at[j]], rows_v.at[b], gsem.at[b])

    def _wait_gather(b):
        pltpu.make_async_copy(xs_sh.at[src_v.at[0]], rows_v.at[b],
                              gsem.at[b]).wait()

    def _start_scatter(j, b):
        pltpu.async_copy(rows_v.at[b], agg_sh.at[dst_v.at[j]], ssem.at[b],
                         add=True)

    def _wait_scatter(b):
        pltpu.make_async_copy(rows_v.at[b], agg_sh.at[dst_v.at[0]],
                              ssem.at[b]).wait()

    for b in range(nbuf):
        _start_gather(b, b)

    @pl.loop(0, NCHUNK - nbuf, step=nbuf)
    def _round(j0):
        for b in range(nbuf):
            _wait_gather(b)
            _start_scatter(j0 + b, b)
            _wait_scatter(b)
            _start_gather(j0 + b + nbuf, b)

    for b in range(nbuf):
        _wait_gather(b)
        _start_scatter(NCHUNK - nbuf + b, b)
        _wait_scatter(b)

    plsc.subcore_barrier()


def _agg32_body(xw_hbm, src_hbm, dst_hbm, dinv_hbm, out_hbm, src_v, dst_v,
                rows_v, agg_sh, xs_sh, gsem, ssem, xw_v, dinv_v):
    c = lax.axis_index("c")
    s = lax.axis_index("s")
    pltpu.sync_copy(src_hbm.at[s], src_v)
    pltpu.sync_copy(dst_hbm.at[s], dst_v)
    _stage_scaled(xw_hbm, dinv_hbm, xw_v, dinv_v, xs_sh, c, s, 16)
    _agg_pipeline(xw_hbm, src_hbm, dst_hbm, src_v, dst_v, rows_v, agg_sh,
                  xs_sh, gsem, ssem, c, s, 16, 8)
    pltpu.sync_copy(agg_sh.at[pl.ds(s * RPT, RPT)],
                    out_hbm.at[pl.ds(s * RPT, RPT), pl.ds(c * 16, 16)])


_agg32_call = pl.kernel(
    _agg32_body,
    out_type=jax.ShapeDtypeStruct((NPAD, 32), jnp.float32),
    mesh=plsc.VectorSubcoreMesh(core_axis_name="c", subcore_axis_name="s"),
    compiler_params=pltpu.CompilerParams(use_tc_tiling_on_sc=False),
    scratch_types=[
        pltpu.VMEM((NCHUNK, CH), jnp.int32),
        pltpu.VMEM((NCHUNK, CH), jnp.int32),
        pltpu.VMEM((8, CH, 16), jnp.float32),
        pltpu.VMEM_SHARED((NPAD, 16), jnp.float32),
        pltpu.VMEM_SHARED((NPAD, 16), jnp.float32),
        pltpu.SemaphoreType.DMA((8,)),
        pltpu.SemaphoreType.DMA((8,)),
        pltpu.VMEM((RPT, 16), jnp.float32),
        pltpu.VMEM((RPT,), jnp.float32),
    ],
)


def _agg64_body(xw2_hbm, src_hbm, dst_hbm, dinv_hbm, b2_hbm,
                batch_hbm, out_hbm, src_v, dst_v, rows_v, agg_sh, xs_sh,
                gsem, ssem, agg_v, xw_v, dinv_v, batch_v, b2_v, acc_v):
    F2 = 32
    c = lax.axis_index("c")
    s = lax.axis_index("s")
    pltpu.sync_copy(src_hbm.at[s], src_v)
    pltpu.sync_copy(dst_hbm.at[s], dst_v)
    pltpu.sync_copy(batch_hbm.at[pl.ds(s * RPT, RPT)], batch_v)
    pltpu.sync_copy(b2_hbm.at[c], b2_v)
    _stage_scaled(xw2_hbm, dinv_hbm, xw_v, dinv_v, xs_sh, c, s, F2)
    _agg_pipeline(xw2_hbm, src_hbm, dst_hbm, src_v, dst_v, rows_v, agg_sh,
                  xs_sh, gsem, ssem, c, s, F2, 4)

    pltpu.sync_copy(agg_sh.at[pl.ds(s * RPT, RPT)], agg_v)

    neg = jnp.full((16,), -jnp.inf, dtype=jnp.float32)

    @pl.loop(0, NGP)
    def _init(g):
        for k in range(F2 // 16):
            acc_v[g, pl.ds(k * 16, 16)] = neg

    b2a = b2_v[pl.ds(0, 16)]
    b2b = b2_v[pl.ds(16, 16)]

    @pl.loop(0, RPT, step=16)
    def _row(r0):
        d16 = dinv_v[pl.ds(r0, 16)]
        g16 = batch_v[pl.ds(r0, 16)]
        for i in range(16):
            r = r0 + i
            d = d16[i]
            g = g16[i]
            ha = d * (agg_v[r, pl.ds(0, 16)] + xw_v[r, pl.ds(0, 16)]) + b2a
            hb = d * (agg_v[r, pl.ds(16, 16)] + xw_v[r, pl.ds(16, 16)]) + b2b
            acc_v[g, pl.ds(0, 16)] = jnp.maximum(acc_v[g, pl.ds(0, 16)], ha)
            acc_v[g, pl.ds(16, 16)] = jnp.maximum(acc_v[g, pl.ds(16, 16)], hb)

    pltpu.sync_copy(acc_v, out_hbm.at[c, s])


_agg64_call = pl.kernel(
    _agg64_body,
    out_type=jax.ShapeDtypeStruct((NC, NS, NGP, 32), jnp.float32),
    mesh=plsc.VectorSubcoreMesh(core_axis_name="c", subcore_axis_name="s"),
    compiler_params=pltpu.CompilerParams(use_tc_tiling_on_sc=False),
    scratch_types=[
        pltpu.VMEM((NCHUNK, CH), jnp.int32),
        pltpu.VMEM((NCHUNK, CH), jnp.int32),
        pltpu.VMEM((4, CH, 32), jnp.float32),
        pltpu.VMEM_SHARED((NPAD, 32), jnp.float32),
        pltpu.VMEM_SHARED((NPAD, 32), jnp.float32),
        pltpu.SemaphoreType.DMA((4,)),
        pltpu.SemaphoreType.DMA((4,)),
        pltpu.VMEM((RPT, 32), jnp.float32),
        pltpu.VMEM((RPT, 32), jnp.float32),
        pltpu.VMEM((RPT,), jnp.float32),
        pltpu.VMEM((RPT,), jnp.int32),
        pltpu.VMEM((32,), jnp.float32),
        pltpu.VMEM((NGP, 32), jnp.float32),
    ],
)


def _tc1_body(x_ref, w1_ref, p_ref, xw1_ref, dinv_ref):
    xw1 = jnp.dot(x_ref[...], w1_ref[...], preferred_element_type=jnp.float32)
    p = p_ref[...]
    deg = p[0] + p[1] + 1.0
    dinv = lax.rsqrt(deg)[:, None]
    row = lax.broadcasted_iota(jnp.int32, (NPAD, 1), 0)
    dinv = jnp.where(row < N, dinv, 0.0)
    xw1_ref[...] = xw1
    dinv_ref[...] = dinv


_tc1 = pl.pallas_call(
    _tc1_body,
    out_shape=(
        jax.ShapeDtypeStruct((NPAD, 32), jnp.float32),
        jax.ShapeDtypeStruct((NPAD, 1), jnp.float32),
    ),
)


def _tc2_body(q_ref, xw1_ref, dinv_ref, b1_ref, w2_ref, xw2_ref):
    agg1 = q_ref[...]
    dinv = dinv_ref[...]
    pre = dinv * agg1 + dinv * dinv * xw1_ref[...] + b1_ref[...]
    h1 = jnp.where(pre >= 0, pre, 0.1 * pre)
    xw2 = jnp.dot(h1, w2_ref[...], preferred_element_type=jnp.float32)
    row = lax.broadcasted_iota(jnp.int32, (NPAD, 1), 0)
    xw2_ref[...] = jnp.where(row < N, xw2, 0.0)


_tc2 = pl.pallas_call(
    _tc2_body,
    out_shape=jax.ShapeDtypeStruct((NPAD, 64), jnp.float32),
)


def _tc3_body(m_ref, l1w_ref, l1b_ref, l2w_ref, l2b_ref, y_ref):
    m = m_ref[...]
    pooled = jnp.concatenate([jnp.max(m[0, :, :NG, :], axis=0),
                              jnp.max(m[1, :, :NG, :], axis=0)], axis=-1)
    t = jnp.dot(pooled, l1w_ref[...], preferred_element_type=jnp.float32) + l1b_ref[...]
    t = jnp.where(t >= 0, t, 0.1 * t)
    y_ref[...] = jnp.dot(t, l2w_ref[...], preferred_element_type=jnp.float32) + l2b_ref[...]


_tc3 = pl.pallas_call(
    _tc3_body,
    out_shape=jax.ShapeDtypeStruct((NG, 32), jnp.float32),
)


def kernel(x, edge_index, batch, W1, b1, W2, b2, L1W, L1b, L2W, L2b):
    edge_index = edge_index.astype(jnp.int32)
    src = edge_index[0]
    dst = edge_index[1]
    pad = jnp.full((EPAD - E,), N, dtype=jnp.int32)
    src_r = jnp.concatenate([src, pad]).reshape(NS, NCHUNK, CH)
    dst_r = jnp.concatenate([dst, pad]).reshape(NS, NCHUNK, CH)
    aux = jnp.concatenate([jnp.ones((CH,), jnp.float32),
                           jnp.zeros((RPT,), jnp.float32)])
    x_pad = jnp.pad(x, ((0, NPAD - N), (0, 0)))
    batch_pad = jnp.concatenate(
        [batch.astype(jnp.int32), jnp.full((NPAD - N,), NG, jnp.int32)])
    b2_r = b2.reshape(NC, 32)

    p = _deg_call(dst_r, aux)
    xw1, dinv = _tc1(x_pad, W1, p)
    d1 = dinv[:, 0]
    q = _agg32_call(xw1, src_r, dst_r, d1)
    xw2 = _tc2(q, xw1, dinv, b1, W2)
    m = _agg64_call(xw2, src_r, dst_r, d1, b2_r, batch_pad)
    y = _tc3(m, L1W, L1b, L2W, L2b)
    return y
